# baseline (speedup 1.0000x reference)
import functools

import jax
import jax.numpy as jnp
from jax import lax
from jax.experimental import pallas as pl
from jax.experimental.pallas import tpu as pltpu
from jax.experimental.pallas import tpu_sc as plsc

N = 10000
E = 320000
C = 128

NC = 2
NS = 16
NW = NC * NS
K = 80
NWIN = 5
WCHUNKS = 25
CHUNKS = NWIN * WCHUNKS
EDGES_PER_W = CHUNKS * K
PAD_E = EDGES_PER_W - E // NW
ROWS_PER_TILE = 632
N_PAD = NS * ROWS_PER_TILE


def _sc_scatter_fn(x_hbm, src_hbm, dst_hbm, zeros_hbm, out_hbm,
                   s0, d0, rows0, rows1, rows2, rows3, acc,
                   semg0, semg1, semg2, semg3):
    cid = lax.axis_index("c")
    sid = lax.axis_index("s")
    w = cid * NS + sid

    def _startg(sw, c, buf, sem):
        pltpu.async_copy(x_hbm.at[sw.at[jnp.asarray(c, jnp.int32)]], buf, sem)

    pltpu.sync_copy(src_hbm.at[w, jnp.int32(0)], s0)
    pltpu.sync_copy(dst_hbm.at[w, jnp.int32(0)], d0)
    _startg(s0, 0, rows0, semg0)
    _startg(s0, 1, rows1, semg1)
    _startg(s0, 2, rows2, semg2)
    pltpu.sync_copy(zeros_hbm, acc.at[pl.ds(sid * ROWS_PER_TILE, ROWS_PER_TILE)])
    plsc.subcore_barrier()

    def _waitg(buf, sem):
        pltpu.make_async_copy(x_hbm.at[pl.ds(0, K)], buf, sem).wait()

    def run_window(sw, dw, primed=False):
        bufs = (rows0, rows1, rows2, rows3)
        sems = (semg0, semg1, semg2, semg3)

        def step(c, j, start_ofs):
            if start_ofs:
                _startg(sw, c + start_ofs, bufs[(j + start_ofs) % 4],
                        sems[(j + start_ofs) % 4])
            _waitg(bufs[j % 4], sems[j % 4])
            pltpu.sync_copy(bufs[j % 4], acc.at[dw.at[c]], add=True)

        if not primed:
            _startg(sw, 0, rows0, semg0)
            _startg(sw, 1, rows1, semg1)
            _startg(sw, 2, rows2, semg2)

        def body(t, c):
            step(c, 0, 3)
            step(c + 1, 1, 3)
            step(c + 2, 2, 3)
            step(c + 3, 3, 3)
            return c + 4

        lax.fori_loop(0, (WCHUNKS - 5) // 4, body, jnp.int32(0))
        t0 = WCHUNKS - 5
        step(jnp.int32(t0), t0 % 4, 3)
        step(jnp.int32(t0 + 1), (t0 + 1) % 4, 3)
        step(jnp.int32(t0 + 2), (t0 + 2) % 4, 0)
        step(jnp.int32(t0 + 3), (t0 + 3) % 4, 0)
        step(jnp.int32(t0 + 4), (t0 + 4) % 4, 0)

    for win in range(NWIN):
        if win >= 1:
            pltpu.sync_copy(src_hbm.at[w, jnp.int32(win)], s0)
            pltpu.sync_copy(dst_hbm.at[w, jnp.int32(win)], d0)
        run_window(s0, d0, primed=(win == 0))

    plsc.subcore_barrier()
    pltpu.sync_copy(acc.at[pl.ds(sid * ROWS_PER_TILE, ROWS_PER_TILE)],
                    out_hbm.at[cid, pl.ds(sid * ROWS_PER_TILE, ROWS_PER_TILE)])


_sc_scatter = functools.partial(
    pl.kernel,
    out_type=jax.ShapeDtypeStruct((NC, N_PAD, C), jnp.float32),
    mesh=plsc.VectorSubcoreMesh(core_axis_name="c", subcore_axis_name="s"),
    scratch_types=[
        pltpu.VMEM((WCHUNKS, K), jnp.int32),
        pltpu.VMEM((WCHUNKS, K), jnp.int32),
        pltpu.VMEM((K, C), jnp.float32),
        pltpu.VMEM((K, C), jnp.float32),
        pltpu.VMEM((K, C), jnp.float32),
        pltpu.VMEM((K, C), jnp.float32),
        pltpu.VMEM_SHARED((N_PAD, C), jnp.float32),
        pltpu.SemaphoreType.DMA,
        pltpu.SemaphoreType.DMA,
        pltpu.SemaphoreType.DMA,
        pltpu.SemaphoreType.DMA,
    ],
)(_sc_scatter_fn)


def _tc_combine_fn(p_ref, w_ref, b_ref, o_ref):
    agg = p_ref[0, :N] + p_ref[1, :N]
    o_ref[...] = jnp.dot(agg, w_ref[...],
                         preferred_element_type=jnp.float32) + b_ref[...]


_tc_combine = pl.pallas_call(
    _tc_combine_fn,
    out_shape=jax.ShapeDtypeStruct((N, C), jnp.float32),
)


def kernel(x, edge_index, edge_weight, W1, b1, W2, b2, a, b):
    src = edge_index[0].astype(jnp.int32).reshape(NW, E // NW)
    dst = edge_index[1].astype(jnp.int32).reshape(NW, E // NW)
    src = src.reshape(NW, NWIN, WCHUNKS, K)
    dst = dst.reshape(NW, NWIN, WCHUNKS, K)
    zeros = jnp.zeros((ROWS_PER_TILE, C), jnp.float32)
    partials = _sc_scatter(x, src, dst, zeros)
    w2t = W2.T.astype(jnp.float32)
    b2_2d = b2.astype(jnp.float32).reshape(1, C)
    return _tc_combine(partials, w2t, b2_2d)


# speedup vs baseline: 1.0252x; 1.0252x over previous
import functools

import jax
import jax.numpy as jnp
from jax import lax
from jax.experimental import pallas as pl
from jax.experimental.pallas import tpu as pltpu
from jax.experimental.pallas import tpu_sc as plsc

N = 10000
E = 320000
C = 128

NC = 2
NS = 16
NW = NC * NS
K = 80
NWIN = 5
WCHUNKS = 25
CHUNKS = NWIN * WCHUNKS
EDGES_PER_W = CHUNKS * K
PAD_E = EDGES_PER_W - E // NW
ROWS_PER_TILE = 632
N_PAD = NS * ROWS_PER_TILE


def _sc_scatter_fn(x_hbm, src_hbm, dst_hbm, zeros_hbm, out_hbm,
                   s0, d0, s1, d1, rows0, rows1, rows2, acc,
                   semg0, semg1, semg2, semi0, semi1):
    cid = lax.axis_index("c")
    sid = lax.axis_index("s")
    w = cid * NS + sid

    def _startg(sw, c, buf, sem):
        pltpu.async_copy(x_hbm.at[sw.at[jnp.asarray(c, jnp.int32)]], buf, sem)

    pltpu.sync_copy(src_hbm.at[w, jnp.int32(0)], s0)
    pltpu.sync_copy(dst_hbm.at[w, jnp.int32(0)], d0)
    _startg(s0, 0, rows0, semg0)
    _startg(s0, 1, rows1, semg1)
    pltpu.async_copy(src_hbm.at[w, jnp.int32(1)], s1, semi1)
    pltpu.async_copy(dst_hbm.at[w, jnp.int32(1)], d1, semi1)
    pltpu.sync_copy(zeros_hbm, acc.at[pl.ds(sid * ROWS_PER_TILE, ROWS_PER_TILE)])
    plsc.subcore_barrier()

    def _waitg(buf, sem):
        pltpu.make_async_copy(x_hbm.at[pl.ds(0, K)], buf, sem).wait()

    def _waiti(bufs, sem):
        pltpu.make_async_copy(src_hbm.at[jnp.int32(0), jnp.int32(0)], bufs[0], sem).wait()
        pltpu.make_async_copy(src_hbm.at[jnp.int32(0), jnp.int32(0)], bufs[1], sem).wait()

    def run_window(sw, dw, primed=False):
        bufs = (rows0, rows1, rows2)
        sems = (semg0, semg1, semg2)

        def step(c, j, start_ofs):
            if start_ofs:
                _startg(sw, c + start_ofs, bufs[(j + start_ofs) % 3],
                        sems[(j + start_ofs) % 3])
            _waitg(bufs[j % 3], sems[j % 3])
            pltpu.sync_copy(bufs[j % 3], acc.at[dw.at[c]], add=True)

        if not primed:
            _startg(sw, 0, rows0, semg0)
            _startg(sw, 1, rows1, semg1)

        def body(t, c):
            step(c, 0, 2)
            step(c + 1, 1, 2)
            step(c + 2, 2, 2)
            return c + 3

        lax.fori_loop(0, (WCHUNKS - 4) // 3, body, jnp.int32(0))
        t0 = WCHUNKS - 4
        step(jnp.int32(t0), t0 % 3, 2)
        step(jnp.int32(t0 + 1), (t0 + 1) % 3, 2)
        step(jnp.int32(t0 + 2), (t0 + 2) % 3, 0)
        step(jnp.int32(t0 + 3), (t0 + 3) % 3, 0)

    for win in range(NWIN):
        bufs = (s0, d0) if win % 2 == 0 else (s1, d1)
        sem = semi0 if win % 2 == 0 else semi1
        if win >= 1:
            _waiti(bufs, sem)
        run_window(*bufs, primed=(win == 0))
        if win + 2 < NWIN:
            pltpu.async_copy(src_hbm.at[w, jnp.int32(win + 2)], bufs[0], sem)
            pltpu.async_copy(dst_hbm.at[w, jnp.int32(win + 2)], bufs[1], sem)

    plsc.subcore_barrier()
    pltpu.sync_copy(acc.at[pl.ds(sid * ROWS_PER_TILE, ROWS_PER_TILE)],
                    out_hbm.at[cid, pl.ds(sid * ROWS_PER_TILE, ROWS_PER_TILE)])


_sc_scatter = functools.partial(
    pl.kernel,
    out_type=jax.ShapeDtypeStruct((NC, N_PAD, C), jnp.float32),
    mesh=plsc.VectorSubcoreMesh(core_axis_name="c", subcore_axis_name="s"),
    scratch_types=[
        pltpu.VMEM((WCHUNKS, K), jnp.int32),
        pltpu.VMEM((WCHUNKS, K), jnp.int32),
        pltpu.VMEM((WCHUNKS, K), jnp.int32),
        pltpu.VMEM((WCHUNKS, K), jnp.int32),
        pltpu.VMEM((K, C), jnp.float32),
        pltpu.VMEM((K, C), jnp.float32),
        pltpu.VMEM((K, C), jnp.float32),
        pltpu.VMEM_SHARED((N_PAD, C), jnp.float32),
        pltpu.SemaphoreType.DMA,
        pltpu.SemaphoreType.DMA,
        pltpu.SemaphoreType.DMA,
        pltpu.SemaphoreType.DMA,
        pltpu.SemaphoreType.DMA,
    ],
)(_sc_scatter_fn)


def _tc_combine_fn(p_ref, w_ref, b_ref, o_ref):
    agg = p_ref[0, :N] + p_ref[1, :N]
    o_ref[...] = jnp.dot(agg, w_ref[...],
                         preferred_element_type=jnp.float32) + b_ref[...]


_tc_combine = pl.pallas_call(
    _tc_combine_fn,
    out_shape=jax.ShapeDtypeStruct((N, C), jnp.float32),
)


def kernel(x, edge_index, edge_weight, W1, b1, W2, b2, a, b):
    src = edge_index[0].astype(jnp.int32).reshape(NW, E // NW)
    dst = edge_index[1].astype(jnp.int32).reshape(NW, E // NW)
    src = src.reshape(NW, NWIN, WCHUNKS, K)
    dst = dst.reshape(NW, NWIN, WCHUNKS, K)
    zeros = jnp.zeros((ROWS_PER_TILE, C), jnp.float32)
    partials = _sc_scatter(x, src, dst, zeros)
    w2t = W2.T.astype(jnp.float32)
    b2_2d = b2.astype(jnp.float32).reshape(1, C)
    return _tc_combine(partials, w2t, b2_2d)
